# baseline (speedup 1.0000x reference)
import functools

import jax
import jax.numpy as jnp
import numpy as np
from jax import lax
from jax.experimental import pallas as pl
from jax.experimental.pallas import tpu as pltpu
from jax.experimental.pallas import tpu_sc as plsc

_VOCAB = 151936
_D = 1024
_B = 2
_S = 2048
_HEAD = 128
_THETA = 1000000.0

_NW = 32
_ROWS_PER_W = (_B * _S) // _NW
_CHUNK = 32
_NCHUNK = _ROWS_PER_W // _CHUNK

_RB = 512
_MINF = float(np.finfo(np.float32).min)

_INV_FREQ = np.concatenate(
    [1.0 / (_THETA ** (np.arange(0, _HEAD, 2) / _HEAD))] * 2
).astype(np.float32).reshape(1, _HEAD)


def _sc_gather_body(ids_hbm, table_hbm, out_hbm, idx_v, bufs, gsem, osem):
    wid = lax.axis_index("s") * 2 + lax.axis_index("c")
    base = wid * _ROWS_PER_W
    pltpu.sync_copy(
        ids_hbm.at[wid // 16, pl.ds((wid % 16) * _ROWS_PER_W, _ROWS_PER_W)], idx_v
    )
    pltpu.async_copy(table_hbm.at[idx_v.at[pl.ds(0, _CHUNK)]], bufs.at[0], gsem)
    for c in range(_NCHUNK):
        cur = bufs.at[c % 2]
        pltpu.make_async_copy(
            table_hbm.at[idx_v.at[pl.ds(c * _CHUNK, _CHUNK)]], cur, gsem
        ).wait()
        if c + 1 < _NCHUNK:
            if c >= 1:
                pltpu.make_async_copy(
                    bufs.at[(c + 1) % 2],
                    out_hbm.at[pl.ds(base + (c - 1) * _CHUNK, _CHUNK)], osem,
                ).wait()
            pltpu.async_copy(
                table_hbm.at[idx_v.at[pl.ds((c + 1) * _CHUNK, _CHUNK)]],
                bufs.at[(c + 1) % 2], gsem,
            )
        pltpu.async_copy(cur, out_hbm.at[pl.ds(base + c * _CHUNK, _CHUNK)], osem)
    for c in (_NCHUNK - 2, _NCHUNK - 1):
        pltpu.make_async_copy(
            bufs.at[c % 2], out_hbm.at[pl.ds(base + c * _CHUNK, _CHUNK)], osem
        ).wait()


@functools.cache
def _sc_gather():
    return pl.kernel(
        _sc_gather_body,
        out_type=jax.ShapeDtypeStruct((_B * _S, _D), jnp.float32),
        mesh=plsc.VectorSubcoreMesh(core_axis_name="c", subcore_axis_name="s"),
        scratch_types=[
            pltpu.VMEM((_ROWS_PER_W,), jnp.int32),
            pltpu.VMEM((2, _CHUNK, _D), jnp.float32),
            pltpu.SemaphoreType.DMA,
            pltpu.SemaphoreType.DMA,
        ],
    )


def _tc_body(inv_ref, mask_ref, cos_ref, sin_ref, pid_ref, cp_ref):
    i = pl.program_id(0)
    rbase = i * _RB
    r = lax.broadcasted_iota(jnp.int32, (_RB, _S), 0) + rbase
    c = lax.broadcasted_iota(jnp.int32, (_RB, _S), 1)
    blk = jnp.where(c <= r, 0.0, _MINF).astype(jnp.float32)
    mask_ref[0, 0] = blk
    mask_ref[1, 0] = blk

    pos = (lax.broadcasted_iota(jnp.int32, (_RB, _HEAD), 0) + rbase).astype(
        jnp.float32)
    f = pos * inv_ref[...]
    cos_ref[0] = jnp.cos(f)
    sin_ref[0] = jnp.sin(f)

    @pl.when(i == 0)
    def _iotas():
        ii = lax.broadcasted_iota(jnp.int32, (1, _S), 1)
        pid_ref[...] = ii
        cp_ref[...] = ii


_tc_call = pl.pallas_call(
    _tc_body,
    grid=(_S // _RB,),
    in_specs=[pl.BlockSpec((1, _HEAD), lambda i: (0, 0))],
    out_specs=(
        pl.BlockSpec((_B, 1, _RB, _S), lambda i: (0, 0, i, 0)),
        pl.BlockSpec((1, _RB, _HEAD), lambda i: (0, i, 0)),
        pl.BlockSpec((1, _RB, _HEAD), lambda i: (0, i, 0)),
        pl.BlockSpec((1, _S), lambda i: (0, 0)),
        pl.BlockSpec((1, _S), lambda i: (0, 0)),
    ),
    out_shape=(
        jax.ShapeDtypeStruct((_B, 1, _S, _S), jnp.float32),
        jax.ShapeDtypeStruct((1, _S, _HEAD), jnp.float32),
        jax.ShapeDtypeStruct((1, _S, _HEAD), jnp.float32),
        jax.ShapeDtypeStruct((1, _S), jnp.int32),
        jax.ShapeDtypeStruct((1, _S), jnp.int32),
    ),
)


@jax.jit
def kernel(input_ids, embed_table):
    hidden = _sc_gather()(input_ids, embed_table).reshape(_B, _S, _D)
    causal_mask, cos, sin, position_ids, cache_position = _tc_call(jnp.asarray(_INV_FREQ))
    return (hidden, causal_mask, position_ids, cache_position.reshape(_S), cos, sin)


# speedup vs baseline: 1.0102x; 1.0102x over previous
import functools

import jax
import jax.numpy as jnp
import numpy as np
from jax import lax
from jax.experimental import pallas as pl
from jax.experimental.pallas import tpu as pltpu
from jax.experimental.pallas import tpu_sc as plsc

_VOCAB = 151936
_D = 1024
_B = 2
_S = 2048
_HEAD = 128
_THETA = 1000000.0

_NW = 32
_ROWS_PER_W = (_B * _S) // _NW
_CHUNK = 32
_NCHUNK = _ROWS_PER_W // _CHUNK

_RB = 256
_MINF = float(np.finfo(np.float32).min)

_INV_FREQ = np.concatenate(
    [1.0 / (_THETA ** (np.arange(0, _HEAD, 2) / _HEAD))] * 2
).astype(np.float32).reshape(1, _HEAD)


def _sc_gather_body(ids_hbm, table_hbm, out_hbm, idx_v, bufs, gsem, osem):
    wid = lax.axis_index("s") * 2 + lax.axis_index("c")
    base = wid * _ROWS_PER_W
    pltpu.sync_copy(
        ids_hbm.at[wid // 16, pl.ds((wid % 16) * _ROWS_PER_W, _ROWS_PER_W)], idx_v
    )
    pltpu.async_copy(table_hbm.at[idx_v.at[pl.ds(0, _CHUNK)]], bufs.at[0], gsem)
    for c in range(_NCHUNK):
        cur = bufs.at[c % 2]
        pltpu.make_async_copy(
            table_hbm.at[idx_v.at[pl.ds(c * _CHUNK, _CHUNK)]], cur, gsem
        ).wait()
        if c + 1 < _NCHUNK:
            if c >= 1:
                pltpu.make_async_copy(
                    bufs.at[(c + 1) % 2],
                    out_hbm.at[pl.ds(base + (c - 1) * _CHUNK, _CHUNK)], osem,
                ).wait()
            pltpu.async_copy(
                table_hbm.at[idx_v.at[pl.ds((c + 1) * _CHUNK, _CHUNK)]],
                bufs.at[(c + 1) % 2], gsem,
            )
        pltpu.async_copy(cur, out_hbm.at[pl.ds(base + c * _CHUNK, _CHUNK)], osem)
    for c in (_NCHUNK - 2, _NCHUNK - 1):
        pltpu.make_async_copy(
            bufs.at[c % 2], out_hbm.at[pl.ds(base + c * _CHUNK, _CHUNK)], osem
        ).wait()


@functools.cache
def _sc_gather():
    return pl.kernel(
        _sc_gather_body,
        out_type=jax.ShapeDtypeStruct((_B * _S, _D), jnp.float32),
        mesh=plsc.VectorSubcoreMesh(core_axis_name="c", subcore_axis_name="s"),
        scratch_types=[
            pltpu.VMEM((_ROWS_PER_W,), jnp.int32),
            pltpu.VMEM((2, _CHUNK, _D), jnp.float32),
            pltpu.SemaphoreType.DMA,
            pltpu.SemaphoreType.DMA,
        ],
    )


def _tc_body(inv_ref, mask_ref, cos_ref, sin_ref, pid_ref, cp_ref):
    i = pl.program_id(0)
    rbase = i * _RB
    r = lax.broadcasted_iota(jnp.int32, (_RB, _S), 0) + rbase
    c = lax.broadcasted_iota(jnp.int32, (_RB, _S), 1)
    blk = jnp.where(c <= r, 0.0, _MINF).astype(jnp.float32)
    mask_ref[0, 0] = blk
    mask_ref[1, 0] = blk

    pos = (lax.broadcasted_iota(jnp.int32, (_RB, _HEAD), 0) + rbase).astype(
        jnp.float32)
    f = pos * inv_ref[...]
    cos_ref[0] = jnp.cos(f)
    sin_ref[0] = jnp.sin(f)

    @pl.when(i == 0)
    def _iotas():
        ii = lax.broadcasted_iota(jnp.int32, (1, _S), 1)
        pid_ref[...] = ii
        cp_ref[...] = ii


_tc_call = pl.pallas_call(
    _tc_body,
    grid=(_S // _RB,),
    in_specs=[pl.BlockSpec((1, _HEAD), lambda i: (0, 0))],
    out_specs=(
        pl.BlockSpec((_B, 1, _RB, _S), lambda i: (0, 0, i, 0)),
        pl.BlockSpec((1, _RB, _HEAD), lambda i: (0, i, 0)),
        pl.BlockSpec((1, _RB, _HEAD), lambda i: (0, i, 0)),
        pl.BlockSpec((1, _S), lambda i: (0, 0)),
        pl.BlockSpec((1, _S), lambda i: (0, 0)),
    ),
    out_shape=(
        jax.ShapeDtypeStruct((_B, 1, _S, _S), jnp.float32),
        jax.ShapeDtypeStruct((1, _S, _HEAD), jnp.float32),
        jax.ShapeDtypeStruct((1, _S, _HEAD), jnp.float32),
        jax.ShapeDtypeStruct((1, _S), jnp.int32),
        jax.ShapeDtypeStruct((1, _S), jnp.int32),
    ),
)


@jax.jit
def kernel(input_ids, embed_table):
    hidden = _sc_gather()(input_ids, embed_table).reshape(_B, _S, _D)
    causal_mask, cos, sin, position_ids, cache_position = _tc_call(jnp.asarray(_INV_FREQ))
    return (hidden, causal_mask, position_ids, cache_position.reshape(_S), cos, sin)
